# Initial kernel scaffold; baseline (speedup 1.0000x reference)
import jax
import jax.numpy as jnp
from jax.experimental import pallas as pl


def kernel(x, edge_index, w_s, w_n):
    raise NotImplementedError("write your pallas kernel here")


# speedup vs baseline: 8.0437x; 8.0437x over previous
import functools

import jax
import jax.numpy as jnp
from jax import lax
from jax.experimental import pallas as pl
from jax.experimental.pallas import tpu as pltpu
from jax.experimental.pallas import tpu_sc as plsc

_BLK = 128
_NC = 2
_NS = 16


def _sc_segment_sum(x, edge_blocks, zeros):
  N, D = x.shape
  NP = zeros.shape[0]
  NB = edge_blocks.shape[0]
  NW = _NC * _NS
  rows_per_tile = NP // _NS
  base, rem = NB // NW, NB % NW

  mesh = plsc.VectorSubcoreMesh(core_axis_name="c", subcore_axis_name="s")

  @functools.partial(
      pl.kernel,
      out_type=jax.ShapeDtypeStruct((_NC, NP, D), jnp.float32),
      mesh=mesh,
      scratch_types=[
          pltpu.VMEM_SHARED((NP, D), jnp.float32),
          pltpu.VMEM((2, _BLK), jnp.int32),
          pltpu.VMEM((_BLK, D), jnp.float32),
          pltpu.SemaphoreType.DMA,
      ],
  )
  def k(x_hbm, eb_hbm, z_hbm, out_hbm, acc, eb_v, rows_v, sem):
    c = lax.axis_index("c")
    s = lax.axis_index("s")
    wid = s * _NC + c
    r0 = s * rows_per_tile

    pltpu.sync_copy(z_hbm.at[pl.ds(r0, rows_per_tile)],
                    acc.at[pl.ds(r0, rows_per_tile)])
    plsc.subcore_barrier()

    nmine = base + jnp.where(wid < rem, 1, 0)

    @pl.loop(0, nmine)
    def _(kk):
      b = wid + NW * kk
      pltpu.sync_copy(eb_hbm.at[b], eb_v)
      pltpu.async_copy(x_hbm.at[eb_v.at[0]], rows_v, sem).wait()
      pltpu.sync_copy(rows_v, acc.at[eb_v.at[1]], add=True)

    plsc.subcore_barrier()
    pltpu.sync_copy(acc.at[pl.ds(r0, rows_per_tile)],
                    out_hbm.at[c, pl.ds(r0, rows_per_tile)])

  return k(x, edge_blocks, zeros)


def _tc_finish(x, partials, w_s, w_n):
  N, D = x.shape
  OUT = w_s.shape[1]
  BN = 1000

  def body(x_ref, p_ref, ws_ref, wn_ref, o_ref):
    agg = p_ref[0] + p_ref[1]
    o_ref[...] = jnp.maximum(
        jnp.dot(x_ref[...], ws_ref[...], preferred_element_type=jnp.float32)
        + jnp.dot(agg, wn_ref[...], preferred_element_type=jnp.float32),
        0.0)

  return pl.pallas_call(
      body,
      grid=(N // BN,),
      in_specs=[
          pl.BlockSpec((BN, D), lambda i: (i, 0)),
          pl.BlockSpec((_NC, BN, D), lambda i: (0, i, 0)),
          pl.BlockSpec((D, OUT), lambda i: (0, 0)),
          pl.BlockSpec((D, OUT), lambda i: (0, 0)),
      ],
      out_specs=pl.BlockSpec((BN, OUT), lambda i: (i, 0)),
      out_shape=jax.ShapeDtypeStruct((N, OUT), jnp.float32),
  )(x, partials, w_s, w_n)


def kernel(x, edge_index, w_s, w_n):
  N, D = x.shape
  E = edge_index.shape[1]
  assert E % _BLK == 0
  align = 8 * _NS
  NP = ((N + align - 1) // align) * align
  edge_blocks = edge_index.reshape(2, E // _BLK, _BLK).transpose(1, 0, 2)
  zeros = jnp.zeros((NP, D), jnp.float32)
  partials = _sc_segment_sum(x, edge_blocks, zeros)
  return _tc_finish(x, partials, w_s, w_n)
